# baseline (speedup 1.0000x reference)
import functools

import jax
import jax.numpy as jnp
from jax import lax
from jax.experimental import pallas as pl
from jax.experimental.pallas import tpu as pltpu
from jax.experimental.pallas import tpu_sc as plsc

N = 10000
E = 320000
IN_FEAT = 128
HID = 32
C = 4
D = HID * C
OUT_FEAT = 128
BETA = 0.1

NC = 2
NS = 16
NW = NC * NS
EPW = E // NW

NPAD = 10240
ZROWS = NPAD // NS

GCH = 1000
NGCH = EPW // GCH
SCH = 200
NSCH = EPW // SCH

RB = 2000
EB = 8000
FCH = 80
NFCH = EPW // FCH
FGRP = FCH // 16

_f32 = jnp.float32


def _mlp_in_body(x_ref, w1_ref, b1_ref, g_ref, be_ref, wl_ref, h_ref, z_ref):
    h = jnp.dot(x_ref[...], w1_ref[...], preferred_element_type=_f32)
    h = jnp.maximum(h + b1_ref[...], 0.0)
    mu = jnp.mean(h, axis=1, keepdims=True)
    var = jnp.mean((h - mu) * (h - mu), axis=1, keepdims=True)
    h = (h - mu) * lax.rsqrt(var + 1e-5) * g_ref[...] + be_ref[...]
    h_ref[...] = h
    z_ref[...] = jnp.dot(h, wl_ref[...], preferred_element_type=_f32)


def _mlp_in(x, w1, b1, g, be, wl):
    grid = N // RB
    return pl.pallas_call(
        _mlp_in_body,
        grid=(grid,),
        in_specs=[
            pl.BlockSpec((RB, IN_FEAT), lambda i: (i, 0)),
            pl.BlockSpec((IN_FEAT, D), lambda i: (0, 0)),
            pl.BlockSpec((1, D), lambda i: (0, 0)),
            pl.BlockSpec((1, D), lambda i: (0, 0)),
            pl.BlockSpec((1, D), lambda i: (0, 0)),
            pl.BlockSpec((D, HID), lambda i: (0, 0)),
        ],
        out_specs=[
            pl.BlockSpec((RB, D), lambda i: (i, 0)),
            pl.BlockSpec((RB, HID), lambda i: (i, 0)),
        ],
        out_shape=[
            jax.ShapeDtypeStruct((N, D), _f32),
            jax.ShapeDtypeStruct((N, HID), _f32),
        ],
    )(x, w1, b1, g, be, wl)


def _edge_body(zr_ref, zc_ref, wa_ref, p_ref, t_ref, j_ref, xj_ref):
    zr = zr_ref[...]
    zc = zc_ref[...]
    b = jnp.maximum(0.5 * zr + zc, 0.0)
    e = jnp.exp(jnp.dot(b, wa_ref[...], preferred_element_type=_f32))
    num = jnp.dot(e, p_ref[...], preferred_element_type=_f32)
    den = jnp.dot(e, j_ref[...], preferred_element_type=_f32)
    zc4 = jnp.dot(zc, t_ref[...], preferred_element_type=_f32)
    xj_ref[...] = num * zc4 / den


def _edge(zr, zc, wa_pad, p_expand, t_tile, j_ones):
    grid = E // EB
    return pl.pallas_call(
        _edge_body,
        grid=(grid,),
        in_specs=[
            pl.BlockSpec((EB, HID), lambda i: (i, 0)),
            pl.BlockSpec((EB, HID), lambda i: (i, 0)),
            pl.BlockSpec((HID, D), lambda i: (0, 0)),
            pl.BlockSpec((D, D), lambda i: (0, 0)),
            pl.BlockSpec((HID, D), lambda i: (0, 0)),
            pl.BlockSpec((D, D), lambda i: (0, 0)),
        ],
        out_specs=pl.BlockSpec((EB, D), lambda i: (i, 0)),
        out_shape=jax.ShapeDtypeStruct((E, D), _f32),
    )(zr, zc, wa_pad, p_expand, t_tile, j_ones)


def _post_mid_body(o_ref, ego_ref, g_ref, be_ref, wl_ref, z_ref):
    t = jnp.maximum(o_ref[0] + o_ref[1], 0.0)
    mu = jnp.mean(t, axis=1, keepdims=True)
    var = jnp.mean((t - mu) * (t - mu), axis=1, keepdims=True)
    t = (t - mu) * lax.rsqrt(var + 1e-5) * g_ref[...] + be_ref[...]
    h = (1.0 - BETA) * t + BETA * ego_ref[...]
    z_ref[...] = jnp.dot(h, wl_ref[...], preferred_element_type=_f32)


def _post_mid(out2, ego, g, be, wl):
    grid = N // RB
    return pl.pallas_call(
        _post_mid_body,
        grid=(grid,),
        in_specs=[
            pl.BlockSpec((2, RB, D), lambda i: (0, i, 0)),
            pl.BlockSpec((RB, D), lambda i: (i, 0)),
            pl.BlockSpec((1, D), lambda i: (0, 0)),
            pl.BlockSpec((1, D), lambda i: (0, 0)),
            pl.BlockSpec((D, HID), lambda i: (0, 0)),
        ],
        out_specs=pl.BlockSpec((RB, HID), lambda i: (i, 0)),
        out_shape=jax.ShapeDtypeStruct((N, HID), _f32),
    )(out2, ego, g, be, wl)


def _post_final_body(o_ref, ego_ref, g_ref, be_ref, w2_ref, b2_ref, y_ref):
    t = jnp.maximum(o_ref[0] + o_ref[1], 0.0)
    mu = jnp.mean(t, axis=1, keepdims=True)
    var = jnp.mean((t - mu) * (t - mu), axis=1, keepdims=True)
    t = (t - mu) * lax.rsqrt(var + 1e-5) * g_ref[...] + be_ref[...]
    h = (1.0 - BETA) * t + BETA * ego_ref[...]
    y_ref[...] = jnp.dot(h, w2_ref[...], preferred_element_type=_f32) + b2_ref[...]


def _post_final(out2, ego, g, be, w2, b2):
    grid = N // RB
    return pl.pallas_call(
        _post_final_body,
        grid=(grid,),
        in_specs=[
            pl.BlockSpec((2, RB, D), lambda i: (0, i, 0)),
            pl.BlockSpec((RB, D), lambda i: (i, 0)),
            pl.BlockSpec((1, D), lambda i: (0, 0)),
            pl.BlockSpec((1, D), lambda i: (0, 0)),
            pl.BlockSpec((D, OUT_FEAT), lambda i: (0, 0)),
            pl.BlockSpec((1, OUT_FEAT), lambda i: (0, 0)),
        ],
        out_specs=pl.BlockSpec((RB, OUT_FEAT), lambda i: (i, 0)),
        out_shape=jax.ShapeDtypeStruct((N, OUT_FEAT), _f32),
    )(out2, ego, g, be, w2, b2)


_sc_mesh = plsc.VectorSubcoreMesh(core_axis_name="c", subcore_axis_name="s")


@functools.partial(
    pl.kernel,
    out_type=(
        jax.ShapeDtypeStruct((E, HID), _f32),
        jax.ShapeDtypeStruct((E, HID), _f32),
    ),
    mesh=_sc_mesh,
    scratch_types=[
        pltpu.VMEM((GCH,), jnp.int32),
        pltpu.VMEM((GCH,), jnp.int32),
        pltpu.VMEM((GCH, HID), _f32),
        pltpu.VMEM((GCH, HID), _f32),
        pltpu.SemaphoreType.DMA,
        pltpu.SemaphoreType.DMA,
    ],
    compiler_params=pltpu.CompilerParams(use_tc_tiling_on_sc=False),
)
def _sc_gather(z_hbm, row_hbm, col_hbm, zr_hbm, zc_hbm,
               ri_v, ci_v, zr_v, zc_v, sem_r, sem_c):
    wid = lax.axis_index("s") * NC + lax.axis_index("c")

    def body(j, carry):
        base = wid * EPW + j * GCH
        pltpu.sync_copy(row_hbm.at[pl.ds(base, GCH)], ri_v)
        pltpu.sync_copy(col_hbm.at[pl.ds(base, GCH)], ci_v)
        cp_r = pltpu.async_copy(z_hbm.at[ri_v], zr_v, sem_r)
        cp_c = pltpu.async_copy(z_hbm.at[ci_v], zc_v, sem_c)
        cp_r.wait()
        cp_c.wait()
        pltpu.sync_copy(zr_v, zr_hbm.at[pl.ds(base, GCH)])
        pltpu.sync_copy(zc_v, zc_hbm.at[pl.ds(base, GCH)])
        return carry

    lax.fori_loop(0, NGCH, body, 0)


@functools.partial(
    pl.kernel,
    out_type=jax.ShapeDtypeStruct((NC, NPAD, D), _f32),
    mesh=_sc_mesh,
    scratch_types=[
        pltpu.VMEM_SHARED((NPAD, D), _f32),
        pltpu.VMEM((SCH,), jnp.int32),
        pltpu.VMEM((SCH, D), _f32),
    ],
    compiler_params=pltpu.CompilerParams(use_tc_tiling_on_sc=False),
)
def _sc_scatter(xj_hbm, seg_hbm, zero_hbm, out_hbm, acc_sh, seg_v, xj_v):
    cid = lax.axis_index("c")
    sid = lax.axis_index("s")
    wid = sid * NC + cid
    pltpu.sync_copy(zero_hbm, acc_sh.at[pl.ds(sid * ZROWS, ZROWS)])
    plsc.subcore_barrier()

    def body(j, carry):
        base = wid * EPW + j * SCH
        pltpu.sync_copy(seg_hbm.at[pl.ds(base, SCH)], seg_v)
        pltpu.sync_copy(xj_hbm.at[pl.ds(base, SCH)], xj_v)
        pltpu.sync_copy(xj_v, acc_sh.at[seg_v], add=True)
        return carry

    lax.fori_loop(0, NSCH, body, 0)
    plsc.subcore_barrier()
    pltpu.sync_copy(acc_sh.at[pl.ds(sid * ZROWS, ZROWS)],
                    out_hbm.at[cid, pl.ds(sid * ZROWS, ZROWS)])


@functools.partial(
    pl.kernel,
    out_type=jax.ShapeDtypeStruct((NC, NPAD, D), _f32),
    mesh=_sc_mesh,
    scratch_types=[
        pltpu.VMEM_SHARED((NPAD, D), _f32),
        pltpu.VMEM((FCH,), jnp.int32),
        pltpu.VMEM((FCH,), jnp.int32),
        pltpu.VMEM((FCH,), jnp.int32),
        pltpu.VMEM((FCH, HID), _f32),
        pltpu.VMEM((FCH, HID), _f32),
        pltpu.VMEM((FCH, D), _f32),
        pltpu.VMEM((HID * C, 16), _f32),
        pltpu.SemaphoreType.DMA,
        pltpu.SemaphoreType.DMA,
    ],
    compiler_params=pltpu.CompilerParams(use_tc_tiling_on_sc=False,
                                         needs_layout_passes=False),
)
def _sc_fused(z_hbm, row_hbm, col_hbm, seg_hbm, zero_hbm, wab_hbm, out_hbm,
              acc_sh, ri_v, ci_v, sg_v, zr_v, zc_v, xj_v, wa_v, sem_a, sem_b):
    cid = lax.axis_index("c")
    sid = lax.axis_index("s")
    wid = sid * NC + cid
    pltpu.sync_copy(zero_hbm, acc_sh.at[pl.ds(sid * ZROWS, ZROWS)])
    pltpu.sync_copy(wab_hbm, wa_v)
    plsc.subcore_barrier()

    lanes = lax.iota(jnp.int32, 16)

    def chunk(j, carry):
        base = wid * EPW + j * FCH
        pltpu.sync_copy(row_hbm.at[pl.ds(base, FCH)], ri_v)
        pltpu.sync_copy(col_hbm.at[pl.ds(base, FCH)], ci_v)
        pltpu.sync_copy(seg_hbm.at[pl.ds(base, FCH)], sg_v)
        cp_a = pltpu.async_copy(z_hbm.at[ri_v], zr_v, sem_a)
        cp_b = pltpu.async_copy(z_hbm.at[ci_v], zc_v, sem_b)
        cp_a.wait()
        cp_b.wait()

        def group(g, carry2):
            idx_e = lanes + g * 16
            zc_d = []
            l = [None] * C
            for d in range(HID):
                cd = jnp.full((16,), d, jnp.int32)
                vr = plsc.load_gather(zr_v, [idx_e, cd])
                vc = plsc.load_gather(zc_v, [idx_e, cd])
                zc_d.append(vc)
                bd = jnp.maximum(0.5 * vr + vc, 0.0)
                for k in range(C):
                    t = bd * wa_v[d * C + k]
                    l[k] = t if l[k] is None else l[k] + t
            ek = [jnp.exp(v) for v in l]
            inv = 1.0 / (ek[0] + ek[1] + ek[2] + ek[3])
            att = [v * inv for v in ek]
            for k in range(C):
                for d in range(HID):
                    cd = jnp.full((16,), k * HID + d, jnp.int32)
                    plsc.store_scatter(xj_v, [idx_e, cd], att[k] * zc_d[d])
            return carry2

        lax.fori_loop(0, FGRP, group, 0)
        pltpu.sync_copy(xj_v, acc_sh.at[sg_v], add=True)
        return carry

    lax.fori_loop(0, NFCH, chunk, 0)
    plsc.subcore_barrier()
    pltpu.sync_copy(acc_sh.at[pl.ds(sid * ZROWS, ZROWS)],
                    out_hbm.at[cid, pl.ds(sid * ZROWS, ZROWS)])


def kernel(x, edge_index, W1, b1, g0, be0, Wl0, Wa0, g1, be1, Wl1, Wa1,
           g2, be2, W2, b2):
    row = edge_index[0]
    col = edge_index[1]
    seg = jnp.where(row == col, jnp.int32(N), row)
    zero = jnp.zeros((ZROWS, D), _f32)
    wa0_pad = jnp.pad(Wa0, ((0, 0), (0, D - C)))
    wa1_pad = jnp.pad(Wa1, ((0, 0), (0, D - C)))
    p_expand = (jnp.arange(D)[:, None] == (jnp.arange(D)[None, :] // HID)
                ).astype(_f32)
    t_tile = (jnp.arange(HID)[:, None] == (jnp.arange(D)[None, :] % HID)
              ).astype(_f32)
    j_ones = jnp.broadcast_to((jnp.arange(D) < C).astype(_f32)[:, None], (D, D))

    h, z = _mlp_in(x, W1, b1.reshape(1, D), g0.reshape(1, D),
                   be0.reshape(1, D), Wl0)
    ego = h

    wab0 = jnp.broadcast_to(Wa0.reshape(-1)[:, None], (HID * C, 16))
    wab1 = jnp.broadcast_to(Wa1.reshape(-1)[:, None], (HID * C, 16))

    out2 = _sc_fused(z, row, col, seg, zero, wab0)
    z = _post_mid(out2, ego, g1.reshape(1, D), be1.reshape(1, D), Wl1)

    out2 = _sc_fused(z, row, col, seg, zero, wab1)
    return _post_final(out2, ego, g2.reshape(1, D), be2.reshape(1, D), W2,
                       b2.reshape(1, OUT_FEAT))


# speedup vs baseline: 1.1728x; 1.1728x over previous
import functools

import jax
import jax.numpy as jnp
from jax import lax
from jax.experimental import pallas as pl
from jax.experimental.pallas import tpu as pltpu
from jax.experimental.pallas import tpu_sc as plsc

N = 10000
E = 320000
IN_FEAT = 128
HID = 32
C = 4
D = HID * C
OUT_FEAT = 128
BETA = 0.1

NC = 2
NS = 16
NW = NC * NS
EPW = E // NW

NPAD = 10240
ZROWS = NPAD // NS

GCH = 1000
NGCH = EPW // GCH
SCH = 200
NSCH = EPW // SCH

RB = 2000
EB = 8000
FCH = 128
NCHT = E // FCH
NPAIR = 39
NEPI = NCHT - NW * 2 * NPAIR
FGRP = FCH // 16

_f32 = jnp.float32


def _mlp_in_body(x_ref, w1_ref, b1_ref, g_ref, be_ref, wl_ref, h_ref, z_ref):
    h = jnp.dot(x_ref[...], w1_ref[...], preferred_element_type=_f32)
    h = jnp.maximum(h + b1_ref[...], 0.0)
    mu = jnp.mean(h, axis=1, keepdims=True)
    var = jnp.mean((h - mu) * (h - mu), axis=1, keepdims=True)
    h = (h - mu) * lax.rsqrt(var + 1e-5) * g_ref[...] + be_ref[...]
    h_ref[...] = h
    z_ref[...] = jnp.dot(h, wl_ref[...], preferred_element_type=_f32)


def _mlp_in(x, w1, b1, g, be, wl):
    grid = N // RB
    return pl.pallas_call(
        _mlp_in_body,
        grid=(grid,),
        in_specs=[
            pl.BlockSpec((RB, IN_FEAT), lambda i: (i, 0)),
            pl.BlockSpec((IN_FEAT, D), lambda i: (0, 0)),
            pl.BlockSpec((1, D), lambda i: (0, 0)),
            pl.BlockSpec((1, D), lambda i: (0, 0)),
            pl.BlockSpec((1, D), lambda i: (0, 0)),
            pl.BlockSpec((D, HID), lambda i: (0, 0)),
        ],
        out_specs=[
            pl.BlockSpec((RB, D), lambda i: (i, 0)),
            pl.BlockSpec((RB, HID), lambda i: (i, 0)),
        ],
        out_shape=[
            jax.ShapeDtypeStruct((N, D), _f32),
            jax.ShapeDtypeStruct((N, HID), _f32),
        ],
    )(x, w1, b1, g, be, wl)


def _edge_body(zr_ref, zc_ref, wa_ref, p_ref, t_ref, j_ref, xj_ref):
    zr = zr_ref[...]
    zc = zc_ref[...]
    b = jnp.maximum(0.5 * zr + zc, 0.0)
    e = jnp.exp(jnp.dot(b, wa_ref[...], preferred_element_type=_f32))
    num = jnp.dot(e, p_ref[...], preferred_element_type=_f32)
    den = jnp.dot(e, j_ref[...], preferred_element_type=_f32)
    zc4 = jnp.dot(zc, t_ref[...], preferred_element_type=_f32)
    xj_ref[...] = num * zc4 / den


def _edge(zr, zc, wa_pad, p_expand, t_tile, j_ones):
    grid = E // EB
    return pl.pallas_call(
        _edge_body,
        grid=(grid,),
        in_specs=[
            pl.BlockSpec((EB, HID), lambda i: (i, 0)),
            pl.BlockSpec((EB, HID), lambda i: (i, 0)),
            pl.BlockSpec((HID, D), lambda i: (0, 0)),
            pl.BlockSpec((D, D), lambda i: (0, 0)),
            pl.BlockSpec((HID, D), lambda i: (0, 0)),
            pl.BlockSpec((D, D), lambda i: (0, 0)),
        ],
        out_specs=pl.BlockSpec((EB, D), lambda i: (i, 0)),
        out_shape=jax.ShapeDtypeStruct((E, D), _f32),
    )(zr, zc, wa_pad, p_expand, t_tile, j_ones)


def _post_mid_body(o_ref, ego_ref, g_ref, be_ref, wl_ref, z_ref):
    t = jnp.maximum(o_ref[0] + o_ref[1], 0.0)
    mu = jnp.mean(t, axis=1, keepdims=True)
    var = jnp.mean((t - mu) * (t - mu), axis=1, keepdims=True)
    t = (t - mu) * lax.rsqrt(var + 1e-5) * g_ref[...] + be_ref[...]
    h = (1.0 - BETA) * t + BETA * ego_ref[...]
    z_ref[...] = jnp.dot(h, wl_ref[...], preferred_element_type=_f32)


def _post_mid(out2, ego, g, be, wl):
    grid = N // RB
    return pl.pallas_call(
        _post_mid_body,
        grid=(grid,),
        in_specs=[
            pl.BlockSpec((2, RB, D), lambda i: (0, i, 0)),
            pl.BlockSpec((RB, D), lambda i: (i, 0)),
            pl.BlockSpec((1, D), lambda i: (0, 0)),
            pl.BlockSpec((1, D), lambda i: (0, 0)),
            pl.BlockSpec((D, HID), lambda i: (0, 0)),
        ],
        out_specs=pl.BlockSpec((RB, HID), lambda i: (i, 0)),
        out_shape=jax.ShapeDtypeStruct((N, HID), _f32),
    )(out2, ego, g, be, wl)


def _post_final_body(o_ref, ego_ref, g_ref, be_ref, w2_ref, b2_ref, y_ref):
    t = jnp.maximum(o_ref[0] + o_ref[1], 0.0)
    mu = jnp.mean(t, axis=1, keepdims=True)
    var = jnp.mean((t - mu) * (t - mu), axis=1, keepdims=True)
    t = (t - mu) * lax.rsqrt(var + 1e-5) * g_ref[...] + be_ref[...]
    h = (1.0 - BETA) * t + BETA * ego_ref[...]
    y_ref[...] = jnp.dot(h, w2_ref[...], preferred_element_type=_f32) + b2_ref[...]


def _post_final(out2, ego, g, be, w2, b2):
    grid = N // RB
    return pl.pallas_call(
        _post_final_body,
        grid=(grid,),
        in_specs=[
            pl.BlockSpec((2, RB, D), lambda i: (0, i, 0)),
            pl.BlockSpec((RB, D), lambda i: (i, 0)),
            pl.BlockSpec((1, D), lambda i: (0, 0)),
            pl.BlockSpec((1, D), lambda i: (0, 0)),
            pl.BlockSpec((D, OUT_FEAT), lambda i: (0, 0)),
            pl.BlockSpec((1, OUT_FEAT), lambda i: (0, 0)),
        ],
        out_specs=pl.BlockSpec((RB, OUT_FEAT), lambda i: (i, 0)),
        out_shape=jax.ShapeDtypeStruct((N, OUT_FEAT), _f32),
    )(out2, ego, g, be, w2, b2)


_sc_mesh = plsc.VectorSubcoreMesh(core_axis_name="c", subcore_axis_name="s")


@functools.partial(
    pl.kernel,
    out_type=(
        jax.ShapeDtypeStruct((E, HID), _f32),
        jax.ShapeDtypeStruct((E, HID), _f32),
    ),
    mesh=_sc_mesh,
    scratch_types=[
        pltpu.VMEM((GCH,), jnp.int32),
        pltpu.VMEM((GCH,), jnp.int32),
        pltpu.VMEM((GCH, HID), _f32),
        pltpu.VMEM((GCH, HID), _f32),
        pltpu.SemaphoreType.DMA,
        pltpu.SemaphoreType.DMA,
    ],
    compiler_params=pltpu.CompilerParams(use_tc_tiling_on_sc=False),
)
def _sc_gather(z_hbm, row_hbm, col_hbm, zr_hbm, zc_hbm,
               ri_v, ci_v, zr_v, zc_v, sem_r, sem_c):
    wid = lax.axis_index("s") * NC + lax.axis_index("c")

    def body(j, carry):
        base = wid * EPW + j * GCH
        pltpu.sync_copy(row_hbm.at[pl.ds(base, GCH)], ri_v)
        pltpu.sync_copy(col_hbm.at[pl.ds(base, GCH)], ci_v)
        cp_r = pltpu.async_copy(z_hbm.at[ri_v], zr_v, sem_r)
        cp_c = pltpu.async_copy(z_hbm.at[ci_v], zc_v, sem_c)
        cp_r.wait()
        cp_c.wait()
        pltpu.sync_copy(zr_v, zr_hbm.at[pl.ds(base, GCH)])
        pltpu.sync_copy(zc_v, zc_hbm.at[pl.ds(base, GCH)])
        return carry

    lax.fori_loop(0, NGCH, body, 0)


@functools.partial(
    pl.kernel,
    out_type=jax.ShapeDtypeStruct((NC, NPAD, D), _f32),
    mesh=_sc_mesh,
    scratch_types=[
        pltpu.VMEM_SHARED((NPAD, D), _f32),
        pltpu.VMEM((SCH,), jnp.int32),
        pltpu.VMEM((SCH, D), _f32),
    ],
    compiler_params=pltpu.CompilerParams(use_tc_tiling_on_sc=False),
)
def _sc_scatter(xj_hbm, seg_hbm, zero_hbm, out_hbm, acc_sh, seg_v, xj_v):
    cid = lax.axis_index("c")
    sid = lax.axis_index("s")
    wid = sid * NC + cid
    pltpu.sync_copy(zero_hbm, acc_sh.at[pl.ds(sid * ZROWS, ZROWS)])
    plsc.subcore_barrier()

    def body(j, carry):
        base = wid * EPW + j * SCH
        pltpu.sync_copy(seg_hbm.at[pl.ds(base, SCH)], seg_v)
        pltpu.sync_copy(xj_hbm.at[pl.ds(base, SCH)], xj_v)
        pltpu.sync_copy(xj_v, acc_sh.at[seg_v], add=True)
        return carry

    lax.fori_loop(0, NSCH, body, 0)
    plsc.subcore_barrier()
    pltpu.sync_copy(acc_sh.at[pl.ds(sid * ZROWS, ZROWS)],
                    out_hbm.at[cid, pl.ds(sid * ZROWS, ZROWS)])


@functools.partial(
    pl.kernel,
    out_type=jax.ShapeDtypeStruct((NC, NPAD, D), _f32),
    mesh=_sc_mesh,
    scratch_types=[
        pltpu.VMEM_SHARED((NPAD, D), _f32),
        pltpu.VMEM((2, FCH), jnp.int32),
        pltpu.VMEM((2, FCH), jnp.int32),
        pltpu.VMEM((FCH, HID), _f32),
        pltpu.VMEM((FCH, HID), _f32),
        pltpu.VMEM((FCH, HID), _f32),
        pltpu.VMEM((FCH, HID), _f32),
        pltpu.VMEM((FCH, D), _f32),
        pltpu.VMEM((FCH,), jnp.int32),
        pltpu.VMEM((HID * C, 16), _f32),
        pltpu.SemaphoreType.DMA,
        pltpu.SemaphoreType.DMA,
        pltpu.SemaphoreType.DMA,
        pltpu.SemaphoreType.DMA,
    ],
    compiler_params=pltpu.CompilerParams(use_tc_tiling_on_sc=False,
                                         needs_layout_passes=False),
)
def _sc_fused(z_hbm, ei_hbm, zero_hbm, wab_hbm, out_hbm,
              acc_sh, rc0, rc1, zr0, zc0, zr1, zc1, xj_v, sg_v, wa_v,
              sa0, sb0, sa1, sb1):
    cid = lax.axis_index("c")
    sid = lax.axis_index("s")
    wid = sid * NC + cid
    pltpu.sync_copy(zero_hbm, acc_sh.at[pl.ds(sid * ZROWS, ZROWS)])
    pltpu.sync_copy(wab_hbm, wa_v)
    plsc.subcore_barrier()

    lanes = lax.iota(jnp.int32, 16)
    z16 = jnp.zeros((16,), _f32)

    def compute_chunk(rc_v, zr_v, zc_v):
        def group(g, carry2):
            idx_e = lanes + g * 16
            zero16 = jnp.full((16,), 0, jnp.int32)
            one16 = jnp.full((16,), 1, jnp.int32)
            rv = plsc.load_gather(rc_v, [zero16, idx_e])
            cv = plsc.load_gather(rc_v, [one16, idx_e])
            plsc.store_scatter(sg_v, [idx_e], rv)
            msk = rv != cv
            zc_d = []
            l = [None] * C
            for d in range(HID):
                cd = jnp.full((16,), d, jnp.int32)
                vr = plsc.load_gather(zr_v, [idx_e, cd])
                vc = plsc.load_gather(zc_v, [idx_e, cd])
                zc_d.append(vc)
                bd = jnp.maximum(0.5 * vr + vc, 0.0)
                for k in range(C):
                    t = bd * wa_v[d * C + k]
                    l[k] = t if l[k] is None else l[k] + t
            ek = [jnp.exp(v) for v in l]
            inv = 1.0 / (ek[0] + ek[1] + ek[2] + ek[3])
            att = [jnp.where(msk, v * inv, z16) for v in ek]
            for k in range(C):
                for d in range(HID):
                    cd2 = jnp.full((16,), k * HID + d, jnp.int32)
                    plsc.store_scatter(xj_v, [idx_e, cd2], att[k] * zc_d[d])
            return carry2

        lax.fori_loop(0, FGRP, group, 0)
        pltpu.sync_copy(xj_v, acc_sh.at[sg_v], add=True)

    def pair(jj, carry):
        c0 = (2 * jj) * NW + wid
        c1 = (2 * jj + 1) * NW + wid
        pltpu.sync_copy(ei_hbm.at[:, pl.ds(c0 * FCH, FCH)], rc0)
        cp_a0 = pltpu.async_copy(z_hbm.at[rc0.at[0]], zr0, sa0)
        cp_b0 = pltpu.async_copy(z_hbm.at[rc0.at[1]], zc0, sb0)
        pltpu.sync_copy(ei_hbm.at[:, pl.ds(c1 * FCH, FCH)], rc1)
        cp_a1 = pltpu.async_copy(z_hbm.at[rc1.at[0]], zr1, sa1)
        cp_b1 = pltpu.async_copy(z_hbm.at[rc1.at[1]], zc1, sb1)
        cp_a0.wait()
        cp_b0.wait()
        compute_chunk(rc0, zr0, zc0)
        cp_a1.wait()
        cp_b1.wait()
        compute_chunk(rc1, zr1, zc1)
        return carry

    lax.fori_loop(0, NPAIR, pair, 0)

    @pl.when(wid < NEPI)
    def _():
        ce = 2 * NPAIR * NW + wid
        pltpu.sync_copy(ei_hbm.at[:, pl.ds(ce * FCH, FCH)], rc0)
        pltpu.async_copy(z_hbm.at[rc0.at[0]], zr0, sa0).wait()
        pltpu.async_copy(z_hbm.at[rc0.at[1]], zc0, sb0).wait()
        compute_chunk(rc0, zr0, zc0)

    plsc.subcore_barrier()
    pltpu.sync_copy(acc_sh.at[pl.ds(sid * ZROWS, ZROWS)],
                    out_hbm.at[cid, pl.ds(sid * ZROWS, ZROWS)])


def kernel(x, edge_index, W1, b1, g0, be0, Wl0, Wa0, g1, be1, Wl1, Wa1,
           g2, be2, W2, b2):
    zero = jnp.zeros((ZROWS, D), _f32)
    wa0_pad = jnp.pad(Wa0, ((0, 0), (0, D - C)))
    wa1_pad = jnp.pad(Wa1, ((0, 0), (0, D - C)))
    p_expand = (jnp.arange(D)[:, None] == (jnp.arange(D)[None, :] // HID)
                ).astype(_f32)
    t_tile = (jnp.arange(HID)[:, None] == (jnp.arange(D)[None, :] % HID)
              ).astype(_f32)
    j_ones = jnp.broadcast_to((jnp.arange(D) < C).astype(_f32)[:, None], (D, D))

    h, z = _mlp_in(x, W1, b1.reshape(1, D), g0.reshape(1, D),
                   be0.reshape(1, D), Wl0)
    ego = h

    wab0 = jnp.broadcast_to(Wa0.reshape(-1)[:, None], (HID * C, 16))
    wab1 = jnp.broadcast_to(Wa1.reshape(-1)[:, None], (HID * C, 16))

    out2 = _sc_fused(z, edge_index, zero, wab0)
    z = _post_mid(out2, ego, g1.reshape(1, D), be1.reshape(1, D), Wl1)

    out2 = _sc_fused(z, edge_index, zero, wab1)
    return _post_final(out2, ego, g2.reshape(1, D), be2.reshape(1, D), W2,
                       b2.reshape(1, OUT_FEAT))


# speedup vs baseline: 1.8942x; 1.6151x over previous
import functools

import jax
import jax.numpy as jnp
from jax import lax
from jax.experimental import pallas as pl
from jax.experimental.pallas import tpu as pltpu
from jax.experimental.pallas import tpu_sc as plsc

N = 10000
E = 320000
IN_FEAT = 128
HID = 32
C = 4
D = HID * C
OUT_FEAT = 128
BETA = 0.1

NC = 2
NS = 16
NW = NC * NS
EPW = E // NW

NPAD = 10240
ZROWS = NPAD // NS

GCH = 1000
NGCH = EPW // GCH
SCH = 200
NSCH = EPW // SCH

RB = 2000
EB = 8000

_f32 = jnp.float32


def _mlp_in_body(x_ref, w1_ref, b1_ref, g_ref, be_ref, wl_ref, h_ref, z_ref):
    h = jnp.dot(x_ref[...], w1_ref[...], preferred_element_type=_f32)
    h = jnp.maximum(h + b1_ref[...], 0.0)
    mu = jnp.mean(h, axis=1, keepdims=True)
    var = jnp.mean((h - mu) * (h - mu), axis=1, keepdims=True)
    h = (h - mu) * lax.rsqrt(var + 1e-5) * g_ref[...] + be_ref[...]
    h_ref[...] = h
    z_ref[...] = jnp.dot(h, wl_ref[...], preferred_element_type=_f32)


def _mlp_in(x, w1, b1, g, be, wl):
    grid = N // RB
    return pl.pallas_call(
        _mlp_in_body,
        grid=(grid,),
        in_specs=[
            pl.BlockSpec((RB, IN_FEAT), lambda i: (i, 0)),
            pl.BlockSpec((IN_FEAT, D), lambda i: (0, 0)),
            pl.BlockSpec((1, D), lambda i: (0, 0)),
            pl.BlockSpec((1, D), lambda i: (0, 0)),
            pl.BlockSpec((1, D), lambda i: (0, 0)),
            pl.BlockSpec((D, HID), lambda i: (0, 0)),
        ],
        out_specs=[
            pl.BlockSpec((RB, D), lambda i: (i, 0)),
            pl.BlockSpec((RB, HID), lambda i: (i, 0)),
        ],
        out_shape=[
            jax.ShapeDtypeStruct((N, D), _f32),
            jax.ShapeDtypeStruct((N, HID), _f32),
        ],
    )(x, w1, b1, g, be, wl)


def _edge_body(zr_ref, zc_ref, wa_ref, p_ref, t_ref, j_ref, xj_ref):
    zr = zr_ref[...]
    zc = zc_ref[...]
    b = jnp.maximum(0.5 * zr + zc, 0.0)
    e = jnp.exp(jnp.dot(b, wa_ref[...], preferred_element_type=_f32))
    num = jnp.dot(e, p_ref[...], preferred_element_type=_f32)
    den = jnp.dot(e, j_ref[...], preferred_element_type=_f32)
    zc4 = jnp.dot(zc, t_ref[...], preferred_element_type=_f32)
    xj_ref[...] = num * zc4 / den


def _edge(zr, zc, wa_pad, p_expand, t_tile, j_ones):
    grid = E // EB
    return pl.pallas_call(
        _edge_body,
        grid=(grid,),
        in_specs=[
            pl.BlockSpec((EB, HID), lambda i: (i, 0)),
            pl.BlockSpec((EB, HID), lambda i: (i, 0)),
            pl.BlockSpec((HID, D), lambda i: (0, 0)),
            pl.BlockSpec((D, D), lambda i: (0, 0)),
            pl.BlockSpec((HID, D), lambda i: (0, 0)),
            pl.BlockSpec((D, D), lambda i: (0, 0)),
        ],
        out_specs=pl.BlockSpec((EB, D), lambda i: (i, 0)),
        out_shape=jax.ShapeDtypeStruct((E, D), _f32),
    )(zr, zc, wa_pad, p_expand, t_tile, j_ones)


def _post_mid_body(o_ref, ego_ref, g_ref, be_ref, wl_ref, z_ref):
    t = jnp.maximum(o_ref[0] + o_ref[1], 0.0)
    mu = jnp.mean(t, axis=1, keepdims=True)
    var = jnp.mean((t - mu) * (t - mu), axis=1, keepdims=True)
    t = (t - mu) * lax.rsqrt(var + 1e-5) * g_ref[...] + be_ref[...]
    h = (1.0 - BETA) * t + BETA * ego_ref[...]
    z_ref[...] = jnp.dot(h, wl_ref[...], preferred_element_type=_f32)


def _post_mid(out2, ego, g, be, wl):
    grid = N // RB
    return pl.pallas_call(
        _post_mid_body,
        grid=(grid,),
        in_specs=[
            pl.BlockSpec((2, RB, D), lambda i: (0, i, 0)),
            pl.BlockSpec((RB, D), lambda i: (i, 0)),
            pl.BlockSpec((1, D), lambda i: (0, 0)),
            pl.BlockSpec((1, D), lambda i: (0, 0)),
            pl.BlockSpec((D, HID), lambda i: (0, 0)),
        ],
        out_specs=pl.BlockSpec((RB, HID), lambda i: (i, 0)),
        out_shape=jax.ShapeDtypeStruct((N, HID), _f32),
    )(out2, ego, g, be, wl)


def _post_final_body(o_ref, ego_ref, g_ref, be_ref, w2_ref, b2_ref, y_ref):
    t = jnp.maximum(o_ref[0] + o_ref[1], 0.0)
    mu = jnp.mean(t, axis=1, keepdims=True)
    var = jnp.mean((t - mu) * (t - mu), axis=1, keepdims=True)
    t = (t - mu) * lax.rsqrt(var + 1e-5) * g_ref[...] + be_ref[...]
    h = (1.0 - BETA) * t + BETA * ego_ref[...]
    y_ref[...] = jnp.dot(h, w2_ref[...], preferred_element_type=_f32) + b2_ref[...]


def _post_final(out2, ego, g, be, w2, b2):
    grid = N // RB
    return pl.pallas_call(
        _post_final_body,
        grid=(grid,),
        in_specs=[
            pl.BlockSpec((2, RB, D), lambda i: (0, i, 0)),
            pl.BlockSpec((RB, D), lambda i: (i, 0)),
            pl.BlockSpec((1, D), lambda i: (0, 0)),
            pl.BlockSpec((1, D), lambda i: (0, 0)),
            pl.BlockSpec((D, OUT_FEAT), lambda i: (0, 0)),
            pl.BlockSpec((1, OUT_FEAT), lambda i: (0, 0)),
        ],
        out_specs=pl.BlockSpec((RB, OUT_FEAT), lambda i: (i, 0)),
        out_shape=jax.ShapeDtypeStruct((N, OUT_FEAT), _f32),
    )(out2, ego, g, be, w2, b2)


_sc_mesh = plsc.VectorSubcoreMesh(core_axis_name="c", subcore_axis_name="s")


@functools.partial(
    pl.kernel,
    out_type=(
        jax.ShapeDtypeStruct((E, HID), _f32),
        jax.ShapeDtypeStruct((E, HID), _f32),
    ),
    mesh=_sc_mesh,
    scratch_types=[
        pltpu.VMEM((GCH,), jnp.int32),
        pltpu.VMEM((GCH,), jnp.int32),
        pltpu.VMEM((GCH, HID), _f32),
        pltpu.VMEM((GCH, HID), _f32),
        pltpu.SemaphoreType.DMA,
        pltpu.SemaphoreType.DMA,
    ],
    compiler_params=pltpu.CompilerParams(use_tc_tiling_on_sc=False),
)
def _sc_gather(z_hbm, row_hbm, col_hbm, zr_hbm, zc_hbm,
               ri_v, ci_v, zr_v, zc_v, sem_r, sem_c):
    wid = lax.axis_index("s") * NC + lax.axis_index("c")

    def body(j, carry):
        base = wid * EPW + j * GCH
        pltpu.sync_copy(row_hbm.at[pl.ds(base, GCH)], ri_v)
        pltpu.sync_copy(col_hbm.at[pl.ds(base, GCH)], ci_v)
        cp_r = pltpu.async_copy(z_hbm.at[ri_v], zr_v, sem_r)
        cp_c = pltpu.async_copy(z_hbm.at[ci_v], zc_v, sem_c)
        cp_r.wait()
        cp_c.wait()
        pltpu.sync_copy(zr_v, zr_hbm.at[pl.ds(base, GCH)])
        pltpu.sync_copy(zc_v, zc_hbm.at[pl.ds(base, GCH)])
        return carry

    lax.fori_loop(0, NGCH, body, 0)


@functools.partial(
    pl.kernel,
    out_type=jax.ShapeDtypeStruct((NC, NPAD, D), _f32),
    mesh=_sc_mesh,
    scratch_types=[
        pltpu.VMEM_SHARED((NPAD, D), _f32),
        pltpu.VMEM((SCH,), jnp.int32),
        pltpu.VMEM((SCH, D), _f32),
    ],
    compiler_params=pltpu.CompilerParams(use_tc_tiling_on_sc=False),
)
def _sc_scatter(xj_hbm, seg_hbm, zero_hbm, out_hbm, acc_sh, seg_v, xj_v):
    cid = lax.axis_index("c")
    sid = lax.axis_index("s")
    wid = sid * NC + cid
    pltpu.sync_copy(zero_hbm, acc_sh.at[pl.ds(sid * ZROWS, ZROWS)])
    plsc.subcore_barrier()

    def body(j, carry):
        base = wid * EPW + j * SCH
        pltpu.sync_copy(seg_hbm.at[pl.ds(base, SCH)], seg_v)
        pltpu.sync_copy(xj_hbm.at[pl.ds(base, SCH)], xj_v)
        pltpu.sync_copy(xj_v, acc_sh.at[seg_v], add=True)
        return carry

    lax.fori_loop(0, NSCH, body, 0)
    plsc.subcore_barrier()
    pltpu.sync_copy(acc_sh.at[pl.ds(sid * ZROWS, ZROWS)],
                    out_hbm.at[cid, pl.ds(sid * ZROWS, ZROWS)])


def kernel(x, edge_index, W1, b1, g0, be0, Wl0, Wa0, g1, be1, Wl1, Wa1,
           g2, be2, W2, b2):
    row = edge_index[0]
    col = edge_index[1]
    seg = jnp.where(row == col, jnp.int32(N), row)
    zero = jnp.zeros((ZROWS, D), _f32)
    wa0_pad = jnp.pad(Wa0, ((0, 0), (0, D - C)))
    wa1_pad = jnp.pad(Wa1, ((0, 0), (0, D - C)))
    p_expand = (jnp.arange(D)[:, None] == (jnp.arange(D)[None, :] // HID)
                ).astype(_f32)
    t_tile = (jnp.arange(HID)[:, None] == (jnp.arange(D)[None, :] % HID)
              ).astype(_f32)
    j_ones = jnp.broadcast_to((jnp.arange(D) < C).astype(_f32)[:, None], (D, D))

    h, z = _mlp_in(x, W1, b1.reshape(1, D), g0.reshape(1, D),
                   be0.reshape(1, D), Wl0)
    ego = h

    zr, zc = _sc_gather(z, row, col)
    xj = _edge(zr, zc, wa0_pad, p_expand, t_tile, j_ones)
    out2 = _sc_scatter(xj, seg, zero)
    z = _post_mid(out2, ego, g1.reshape(1, D), be1.reshape(1, D), Wl1)

    zr, zc = _sc_gather(z, row, col)
    xj = _edge(zr, zc, wa1_pad, p_expand, t_tile, j_ones)
    out2 = _sc_scatter(xj, seg, zero)
    return _post_final(out2, ego, g2.reshape(1, D), be2.reshape(1, D), W2,
                       b2.reshape(1, OUT_FEAT))
